# baseline (speedup 1.0000x reference)
import functools
import jax
import jax.numpy as jnp
from jax import lax
from jax.experimental import pallas as pl
from jax.experimental.pallas import tpu as pltpu
from jax.experimental.pallas import tpu_sc as plsc

VOCAB = 32768
MASK = 32767
N_ROWS = 2048
LANES = 16
NW = 32
ROWS_PER_W = N_ROWS // NW
ELEMS_PER_W = ROWS_PER_W * VOCAB
ZELEMS = 65536
NSLOT = 2 * ROWS_PER_W


def _sc_body(ids_hbm, ls_hbm, z_hbm, out_hbm, ids_v, ls_v, z_v, val_v, idx_v, sem):
    wid = lax.axis_index("s") * 2 + lax.axis_index("c")
    row0 = wid * ROWS_PER_W

    pltpu.sync_copy(ids_hbm.at[pl.ds(row0, ROWS_PER_W)], ids_v)
    pltpu.sync_copy(ls_hbm.at[pl.ds(row0, ROWS_PER_W)], ls_v)
    pltpu.sync_copy(z_hbm, z_v)

    one = jnp.full((LANES,), 1.0, jnp.float32)
    for c in range(ROWS_PER_W // LANES):
        ids16 = ids_v[pl.ds(c * LANES, LANES)]
        ls16 = ls_v[pl.ds(c * LANES, LANES)]
        x = jnp.minimum(jnp.maximum(ls16, -10.0), 10.0)
        alpha = 1.0 / (1.0 + jnp.exp(-x))
        is_mask = ids16 == jnp.full((LANES,), MASK, jnp.int32)
        row = (jnp.full((LANES,), row0 + c * LANES, jnp.int32)
               + lax.iota(jnp.int32, LANES)) * VOCAB
        val_v[pl.ds(c * 2 * LANES, LANES)] = jnp.where(is_mask, one, alpha)
        val_v[pl.ds((c * 2 + 1) * LANES, LANES)] = jnp.where(is_mask, one, 1.0 - alpha)
        idx_v[pl.ds(c * 2 * LANES, LANES)] = row + ids16
        idx_v[pl.ds((c * 2 + 1) * LANES, LANES)] = row + jnp.full((LANES,), MASK, jnp.int32)

    e0 = wid * ELEMS_PER_W
    for k in range(ELEMS_PER_W // ZELEMS):
        pltpu.sync_copy(z_v, out_hbm.at[pl.ds(e0 + k * ZELEMS, ZELEMS)])

    pltpu.async_copy(val_v, out_hbm.at[idx_v], sem).wait()


@jax.jit
def _run(ids_flat, ls_flat, zeros_hbm):
    mesh = plsc.VectorSubcoreMesh(
        core_axis_name="c", subcore_axis_name="s", num_cores=2, num_subcores=16)
    k = functools.partial(
        pl.kernel,
        mesh=mesh,
        out_type=jax.ShapeDtypeStruct((N_ROWS * VOCAB,), jnp.float32),
        scratch_types=[
            pltpu.VMEM((ROWS_PER_W,), jnp.int32),
            pltpu.VMEM((ROWS_PER_W,), jnp.float32),
            pltpu.VMEM((ZELEMS,), jnp.float32),
            pltpu.VMEM((NSLOT,), jnp.float32),
            pltpu.VMEM((NSLOT,), jnp.int32),
            pltpu.SemaphoreType.DMA,
        ],
    )(_sc_body)
    return k(ids_flat, ls_flat, zeros_hbm)


def kernel(log_snr, input_ids):
    B, L = log_snr.shape
    ids_flat = input_ids.astype(jnp.int32).reshape(-1)
    ls_flat = log_snr.reshape(-1)
    zeros_hbm = jnp.zeros((ZELEMS,), jnp.float32)
    out = _run(ids_flat, ls_flat, zeros_hbm)
    return out.reshape(B, L, VOCAB)


# speedup vs baseline: 1.0170x; 1.0170x over previous
import functools
import jax
import jax.numpy as jnp
from jax import lax
from jax.experimental import pallas as pl
from jax.experimental.pallas import tpu as pltpu
from jax.experimental.pallas import tpu_sc as plsc

VOCAB = 32768
MASK = 32767
N_ROWS = 2048
LANES = 16
NW = 32
ROWS_PER_W = N_ROWS // NW
NSLOT = 2 * ROWS_PER_W
BV = 4096


def _zero_block(out_ref):
    out_ref[...] = jnp.zeros_like(out_ref)


def _sc_scatter(ids_hbm, ls_hbm, out_ref, ids_v, ls_v, val_v, idx_v, sem):
    wid = lax.axis_index("s") * 2 + lax.axis_index("c")
    row0 = wid * ROWS_PER_W

    pltpu.sync_copy(ids_hbm.at[pl.ds(row0, ROWS_PER_W)], ids_v)
    pltpu.sync_copy(ls_hbm.at[pl.ds(row0, ROWS_PER_W)], ls_v)

    one = jnp.full((LANES,), 1.0, jnp.float32)
    for c in range(ROWS_PER_W // LANES):
        ids16 = ids_v[pl.ds(c * LANES, LANES)]
        ls16 = ls_v[pl.ds(c * LANES, LANES)]
        x = jnp.minimum(jnp.maximum(ls16, -10.0), 10.0)
        alpha = 1.0 / (1.0 + jnp.exp(-x))
        is_mask = ids16 == jnp.full((LANES,), MASK, jnp.int32)
        row = (jnp.full((LANES,), row0 + c * LANES, jnp.int32)
               + lax.iota(jnp.int32, LANES)) * VOCAB
        val_v[pl.ds(c * 2 * LANES, LANES)] = jnp.where(is_mask, one, alpha)
        val_v[pl.ds((c * 2 + 1) * LANES, LANES)] = jnp.where(is_mask, one, 1.0 - alpha)
        idx_v[pl.ds(c * 2 * LANES, LANES)] = row + ids16
        idx_v[pl.ds((c * 2 + 1) * LANES, LANES)] = row + jnp.full(
            (LANES,), MASK, jnp.int32)

    pltpu.async_copy(val_v, out_ref.at[idx_v], sem).wait()


def kernel(log_snr, input_ids):
    B, L = log_snr.shape
    ids_flat = input_ids.astype(jnp.int32).reshape(-1)
    ls_flat = log_snr.reshape(-1)

    zeros = pl.pallas_call(
        _zero_block,
        grid=(B * L * VOCAB // (512 * BV), 1),
        out_specs=pl.BlockSpec((512 * BV,), lambda i, j: (i,)),
        out_shape=jax.ShapeDtypeStruct((N_ROWS * VOCAB,), jnp.float32),
        compiler_params=pltpu.CompilerParams(
            dimension_semantics=("arbitrary", "arbitrary"),
        ),
    )()

    mesh = plsc.VectorSubcoreMesh(
        core_axis_name="c", subcore_axis_name="s", num_cores=2, num_subcores=16)
    sc_k = functools.partial(
        pl.kernel,
        mesh=mesh,
        out_type=(),
        scratch_types=[
            pltpu.VMEM((ROWS_PER_W,), jnp.int32),
            pltpu.VMEM((ROWS_PER_W,), jnp.float32),
            pltpu.VMEM((NSLOT,), jnp.float32),
            pltpu.VMEM((NSLOT,), jnp.int32),
            pltpu.SemaphoreType.DMA,
        ],
    )(_sc_scatter)

    out_ref = jax.new_ref(zeros)
    sc_k(ids_flat, ls_flat, out_ref)
    return out_ref[...].reshape(B, L, VOCAB)


# speedup vs baseline: 1.0194x; 1.0024x over previous
import functools
import jax
import jax.numpy as jnp
from jax import lax
from jax.experimental import pallas as pl
from jax.experimental.pallas import tpu as pltpu
from jax.experimental.pallas import tpu_sc as plsc

VOCAB = 32768
MASK = 32767
N_ROWS = 2048
LANES = 16
NW = 32
ROWS_PER_W = N_ROWS // NW
NSLOT = 2 * ROWS_PER_W
BV = 4096


def _zero_block(out_ref):
    out_ref[...] = jnp.zeros_like(out_ref)


def _sc_scatter(ids_hbm, ls_hbm, out_ref, ids_v, ls_v, val_v, idx_v, sem):
    wid = lax.axis_index("s") * 2 + lax.axis_index("c")
    row0 = wid * ROWS_PER_W

    pltpu.sync_copy(ids_hbm.at[pl.ds(row0, ROWS_PER_W)], ids_v)
    pltpu.sync_copy(ls_hbm.at[pl.ds(row0, ROWS_PER_W)], ls_v)

    one = jnp.full((LANES,), 1.0, jnp.float32)
    for c in range(ROWS_PER_W // LANES):
        ids16 = ids_v[pl.ds(c * LANES, LANES)]
        ls16 = ls_v[pl.ds(c * LANES, LANES)]
        x = jnp.minimum(jnp.maximum(ls16, -10.0), 10.0)
        alpha = 1.0 / (1.0 + jnp.exp(-x))
        is_mask = ids16 == jnp.full((LANES,), MASK, jnp.int32)
        row = (jnp.full((LANES,), row0 + c * LANES, jnp.int32)
               + lax.iota(jnp.int32, LANES)) * VOCAB
        val_v[pl.ds(c * 2 * LANES, LANES)] = jnp.where(is_mask, one, alpha)
        val_v[pl.ds((c * 2 + 1) * LANES, LANES)] = jnp.where(is_mask, one, 1.0 - alpha)
        idx_v[pl.ds(c * 2 * LANES, LANES)] = row + ids16
        idx_v[pl.ds((c * 2 + 1) * LANES, LANES)] = row + jnp.full(
            (LANES,), MASK, jnp.int32)

    pltpu.async_copy(val_v, out_ref.at[idx_v], sem).wait()


def kernel(log_snr, input_ids):
    B, L = log_snr.shape
    ids_flat = input_ids.astype(jnp.int32).reshape(-1)
    ls_flat = log_snr.reshape(-1)

    zeros = pl.pallas_call(
        _zero_block,
        grid=(B * L * VOCAB // (512 * BV), 1),
        out_specs=pl.BlockSpec((512 * BV,), lambda i, j: (i,)),
        out_shape=jax.ShapeDtypeStruct((N_ROWS * VOCAB,), jnp.float32),
        compiler_params=pltpu.CompilerParams(
            dimension_semantics=("arbitrary", "arbitrary"),
        ),
    )()

    mesh = plsc.VectorSubcoreMesh(
        core_axis_name="c", subcore_axis_name="s", num_cores=2, num_subcores=16)
    sc_k = functools.partial(
        pl.kernel,
        mesh=mesh,
        out_type=(),
        scratch_types=[
            pltpu.VMEM((ROWS_PER_W,), jnp.int32),
            pltpu.VMEM((ROWS_PER_W,), jnp.float32),
            pltpu.VMEM((NSLOT,), jnp.float32),
            pltpu.VMEM((NSLOT,), jnp.int32),
            pltpu.SemaphoreType.DMA,
        ],
    )(_sc_scatter)

    out_ref = jax.new_ref(zeros)
    sc_k(ids_flat, ls_flat, out_ref)
    return jax.freeze(out_ref).reshape(B, L, VOCAB)
